# Initial kernel scaffold; baseline (speedup 1.0000x reference)
import jax
import jax.numpy as jnp
from jax.experimental import pallas as pl


def kernel(x, edge_index, batch, tfidf_vec, ggnn_weight, w_ih, w_hh, b_ih, b_hh, fc1_w, fc1_b, fc2_w, fc2_b):
    raise NotImplementedError("write your pallas kernel here")


# speedup vs baseline: 31.2157x; 31.2157x over previous
import functools

import jax
import jax.numpy as jnp
from jax import lax
from jax.experimental import pallas as pl
from jax.experimental.pallas import tpu as pltpu
from jax.experimental.pallas import tpu_sc as plsc

_N = 100000
_E = 3200000
_D1 = 16
_G = 64
_NC = 6

_NCORES = 2
_NSUB = 16
_NW = _NCORES * _NSUB
_GB = 128
_GPW = 784
_TOTG = _NW * _GPW
_EPAD = _TOTG * _GB
_KG = 28
_NOUTER = _GPW // _KG

_NPAD = 100096
_TROWS = _NPAD // _NSUB

_R = 4352
_NBLK = _NPAD // _R


def _sc_edge_pass(with_deg):
  mesh = plsc.VectorSubcoreMesh(core_axis_name="c", subcore_axis_name="s")
  out_type = [jax.ShapeDtypeStruct((_NCORES, _NPAD, _D1), jnp.float32)]
  if with_deg:
    out_type.append(jax.ShapeDtypeStruct((_NCORES * _NPAD,), jnp.float32))
  scratch = [
      pltpu.VMEM((_KG, _GB), jnp.int32),
      pltpu.VMEM((_KG, _GB), jnp.int32),
      pltpu.VMEM((4 * _GB, _D1), jnp.float32),
      pltpu.VMEM((_GB,), jnp.float32),
      pltpu.VMEM((_GB, _D1), jnp.float32),
      pltpu.VMEM((_GB, _D1), jnp.float32),
      pltpu.VMEM((1024,), jnp.float32),
      pltpu.VMEM_SHARED((_NPAD, _D1), jnp.float32),
      pltpu.VMEM_SHARED((_NPAD,), jnp.float32),
      pltpu.SemaphoreType.DMA,
      pltpu.SemaphoreType.DMA,
  ]
  _NCH = 49
  _CTAIL = _TROWS - 48 * _GB

  def body(m_hbm, src_hbm, dst_hbm, z2_hbm, z1_hbm, ones_hbm, *rest):
    if with_deg:
      out_hbm, deg_hbm = rest[0], rest[1]
      scr = rest[2:]
    else:
      out_hbm = rest[0]
      scr = rest[1:]
    (src_v, dst_v, rows, ones_v, stage_a, stage_b, deg_v, acc_sh, deg_sh,
     sem_a, sem_b) = scr
    c = lax.axis_index("c")
    s = lax.axis_index("s")
    wid = c * _NSUB + s

    row0 = s * _TROWS
    pltpu.sync_copy(z2_hbm, stage_a)
    for k in range(48):
      pltpu.sync_copy(stage_a, acc_sh.at[pl.ds(row0 + k * _GB, _GB)])
    pltpu.sync_copy(stage_a.at[pl.ds(0, _CTAIL)],
                    acc_sh.at[pl.ds(row0 + 48 * _GB, _CTAIL)])
    if with_deg:
      pltpu.sync_copy(z1_hbm, deg_v)
      for k in range(6):
        pltpu.sync_copy(deg_v, deg_sh.at[pl.ds(row0 + k * 1024, 1024)])
      pltpu.sync_copy(deg_v.at[pl.ds(0, _CTAIL)],
                      deg_sh.at[pl.ds(row0 + 6144, _CTAIL)])
    pltpu.sync_copy(ones_hbm, ones_v)
    plsc.subcore_barrier()

    sems = (sem_a, sem_b)
    g_base = wid * _GPW

    def fire_pair(first_g, half, sem):
      for b in range(2):
        pltpu.async_copy(
            m_hbm.at[src_v.at[first_g + b]],
            rows.at[pl.ds((half * 2 + b) * _GB, _GB)],
            sem,
        )

    def outer(blk, carry):
      g0 = g_base + blk * _KG
      pltpu.sync_copy(src_hbm.at[pl.ds(g0, _KG)], src_v)
      pltpu.sync_copy(dst_hbm.at[pl.ds(g0, _KG)], dst_v)
      fire_pair(0, 0, sems[0])

      def inner(ci, carry2):
        for half in range(2):
          qbase = ci * 4 + half * 2
          nbase = qbase + 2

          @pl.when(nbase < _KG)
          def _():
            fire_pair(nbase, 1 - half, sems[1 - half])

          pltpu.make_async_copy(
              m_hbm.at[pl.ds(0, 2 * _GB)],
              rows.at[pl.ds(half * 2 * _GB, 2 * _GB)],
              sems[half],
          ).wait()
          for b in range(2):
            g = qbase + b
            rsl = rows.at[pl.ds((half * 2 + b) * _GB, _GB)]
            pltpu.sync_copy(rsl, acc_sh.at[dst_v.at[g]], add=True)
            if with_deg:
              pltpu.sync_copy(ones_v, deg_sh.at[dst_v.at[g]], add=True)
        return carry2

      lax.fori_loop(0, _KG // 4, inner, 0)
      return carry

    lax.fori_loop(0, _NOUTER, outer, 0)
    plsc.subcore_barrier()

    for k in range(_NCH):
      nrows = _GB if k < _NCH - 1 else _CTAIL
      buf = stage_a if k % 2 == 0 else stage_b
      sem = sems[k % 2]
      off = row0 + k * _GB
      if k >= 2:
        pltpu.make_async_copy(
            buf, out_hbm.at[c, pl.ds(row0, _GB)], sem).wait()
      pltpu.sync_copy(acc_sh.at[pl.ds(off, nrows)], buf.at[pl.ds(0, nrows)])
      pltpu.async_copy(buf.at[pl.ds(0, nrows)],
                       out_hbm.at[c, pl.ds(off, nrows)], sem)
    pltpu.make_async_copy(
        stage_b, out_hbm.at[c, pl.ds(row0, _GB)], sems[1]).wait()
    pltpu.make_async_copy(
        stage_a.at[pl.ds(0, _CTAIL)], out_hbm.at[c, pl.ds(row0, _CTAIL)],
        sems[0]).wait()
    if with_deg:
      for k in range(6):
        pltpu.sync_copy(deg_sh.at[pl.ds(row0 + k * 1024, 1024)], deg_v)
        pltpu.sync_copy(
            deg_v, deg_hbm.at[pl.ds(c * _NPAD + row0 + k * 1024, 1024)])
      pltpu.sync_copy(deg_sh.at[pl.ds(row0 + 6144, _CTAIL)],
                      deg_v.at[pl.ds(0, _CTAIL)])
      pltpu.sync_copy(deg_v.at[pl.ds(0, _CTAIL)],
                      deg_hbm.at[pl.ds(c * _NPAD + row0 + 6144, _CTAIL)])

  return pl.kernel(
      body, mesh=mesh, out_type=out_type, scratch_types=scratch,
      compiler_params=pltpu.CompilerParams(use_tc_tiling_on_sc=False))


def _mm_kernel(x_ref, w_ref, o_ref):
  o_ref[...] = jnp.dot(x_ref[...], w_ref[...],
                       preferred_element_type=jnp.float32)


def _matmul_nd(x, w):
  return pl.pallas_call(
      _mm_kernel,
      grid=(_NBLK,),
      in_specs=[
          pl.BlockSpec((_R, _D1), lambda i: (i, 0)),
          pl.BlockSpec((_D1, _D1), lambda i: (0, 0)),
      ],
      out_specs=pl.BlockSpec((_R, _D1), lambda i: (i, 0)),
      out_shape=jax.ShapeDtypeStruct((_NPAD, _D1), jnp.float32),
  )(x, w)


def _gru_block(agg, h, w):
  (wir, wiz, win, whr, whz, whn, bir, biz, bin_, bhr, bhz, bhn) = w
  dot = functools.partial(jnp.dot, preferred_element_type=jnp.float32)
  r = jax.nn.sigmoid(dot(agg, wir) + bir + dot(h, whr) + bhr)
  z = jax.nn.sigmoid(dot(agg, wiz) + biz + dot(h, whz) + bhz)
  n = jnp.tanh(dot(agg, win) + bin_ + r * (dot(h, whn) + bhn))
  return (1.0 - z) * n + z * h


_GRU_W_SPECS = (
    [pl.BlockSpec((_D1, _D1), lambda i: (0, 0))] * 6
    + [pl.BlockSpec((1, _D1), lambda i: (0, 0))] * 6
)


def _gru1_kernel(s_ref, d_ref, x_ref, *rest):
  w = [r[...] for r in rest[:12]]
  w1 = rest[12]
  h1_ref, m2_ref, dc_ref = rest[13], rest[14], rest[15]
  dc = jnp.maximum(d_ref[0] + d_ref[1], 1.0)
  agg = (s_ref[0] + s_ref[1]) / dc
  h1 = _gru_block(agg, x_ref[...], w)
  h1_ref[...] = h1
  m2_ref[...] = jnp.dot(h1, w1[...], preferred_element_type=jnp.float32)
  dc_ref[...] = dc


def _gru1_call(s1p, degp, x, gru_w, w1):
  return pl.pallas_call(
      _gru1_kernel,
      grid=(_NBLK,),
      in_specs=[
          pl.BlockSpec((_NCORES, _R, _D1), lambda i: (0, i, 0)),
          pl.BlockSpec((_NCORES, _R, 1), lambda i: (0, i, 0)),
          pl.BlockSpec((_R, _D1), lambda i: (i, 0)),
      ] + _GRU_W_SPECS + [pl.BlockSpec((_D1, _D1), lambda i: (0, 0))],
      out_specs=[
          pl.BlockSpec((_R, _D1), lambda i: (i, 0)),
          pl.BlockSpec((_R, _D1), lambda i: (i, 0)),
          pl.BlockSpec((_R, 1), lambda i: (i, 0)),
      ],
      out_shape=[
          jax.ShapeDtypeStruct((_NPAD, _D1), jnp.float32),
          jax.ShapeDtypeStruct((_NPAD, _D1), jnp.float32),
          jax.ShapeDtypeStruct((_NPAD, 1), jnp.float32),
      ],
  )(s1p, degp, x, *gru_w, w1)


def _gru2_kernel(s_ref, dc_ref, h_ref, b3_ref, tfidf_ref, f1a_ref, f1b_ref,
                 f1bias_ref, f2w_ref, f2b_ref, *rest):
  w = [r[...] for r in rest[:12]]
  out_ref, accp, accc = rest[12], rest[13], rest[14]
  i = pl.program_id(0)
  agg = (s_ref[0] + s_ref[1]) / dc_ref[...]
  h2 = _gru_block(agg, h_ref[...], w)
  hr = jnp.maximum(h2, 0.0)
  bvec = b3_ref[0]
  oh = (lax.broadcasted_iota(jnp.int32, (_G, _R), 0) == bvec)
  ohf = oh.astype(jnp.float32)
  pp = jnp.dot(ohf, hr, preferred_element_type=jnp.float32)
  pc = jnp.sum(ohf, axis=1, keepdims=True)

  @pl.when(i == 0)
  def _():
    accp[...] = jnp.zeros_like(accp)
    accc[...] = jnp.zeros_like(accc)

  accp[...] += pp
  accc[...] += pc + jnp.zeros((_G, _D1), jnp.float32)

  @pl.when(i == _NBLK - 1)
  def _():
    dot = functools.partial(jnp.dot, preferred_element_type=jnp.float32)
    mean = accp[...] / jnp.maximum(accc[...], 1.0)
    a = jnp.maximum(
        dot(mean, f1a_ref[...]) + dot(tfidf_ref[...], f1b_ref[...])
        + f1bias_ref[...], 0.0)
    logits = dot(a, f2w_ref[...]) + f2b_ref[...]
    mx = jnp.max(logits, axis=1, keepdims=True)
    lse = jnp.log(jnp.sum(jnp.exp(logits - mx), axis=1, keepdims=True)) + mx
    out_ref[...] = logits - lse


def _gru2_call(s2p, degc, h1, batch3, tfidf, f1a, f1b, f1bias, f2w, f2b,
               gru_w):
  return pl.pallas_call(
      _gru2_kernel,
      grid=(_NBLK,),
      in_specs=[
          pl.BlockSpec((_NCORES, _R, _D1), lambda i: (0, i, 0)),
          pl.BlockSpec((_R, 1), lambda i: (i, 0)),
          pl.BlockSpec((_R, _D1), lambda i: (i, 0)),
          pl.BlockSpec((1, 1, _R), lambda i: (i, 0, 0)),
          pl.BlockSpec((_G, _G), lambda i: (0, 0)),
          pl.BlockSpec((_D1, _G), lambda i: (0, 0)),
          pl.BlockSpec((_G, _G), lambda i: (0, 0)),
          pl.BlockSpec((1, _G), lambda i: (0, 0)),
          pl.BlockSpec((_G, _NC), lambda i: (0, 0)),
          pl.BlockSpec((1, _NC), lambda i: (0, 0)),
      ] + _GRU_W_SPECS,
      out_specs=pl.BlockSpec((_G, _NC), lambda i: (0, 0)),
      out_shape=jax.ShapeDtypeStruct((_G, _NC), jnp.float32),
      scratch_shapes=[
          pltpu.VMEM((_G, _D1), jnp.float32),
          pltpu.VMEM((_G, _D1), jnp.float32),
      ],
  )(s2p, degc, h1, batch3, tfidf, f1a, f1b, f1bias, f2w, f2b, *gru_w)


def kernel(x, edge_index, batch, tfidf_vec, ggnn_weight, w_ih, w_hh, b_ih,
           b_hh, fc1_w, fc1_b, fc2_w, fc2_b):
  f32 = jnp.float32
  src = edge_index[0]
  dst = edge_index[1]
  pad = _EPAD - _E
  src_p = jnp.concatenate([src, jnp.zeros((pad,), jnp.int32)]).reshape(
      _TOTG, _GB)
  dst_p = jnp.concatenate(
      [dst, _N + (jnp.arange(pad, dtype=jnp.int32) % (_NPAD - _N))]).reshape(
          _TOTG, _GB)
  zeros2d = jnp.zeros((_GB, _D1), f32)
  zeros1d = jnp.zeros((1024,), f32)
  ones128 = jnp.ones((_GB,), f32)
  x_p = jnp.concatenate([x, jnp.zeros((_NPAD - _N, _D1), f32)], axis=0)
  batch3 = jnp.concatenate(
      [batch, jnp.full((_NPAD - _N,), _G, jnp.int32)]).reshape(_NBLK, 1, _R)

  wi = w_ih.T
  wh = w_hh.T
  gru_w = (
      wi[:, 0:16], wi[:, 16:32], wi[:, 32:48],
      wh[:, 0:16], wh[:, 16:32], wh[:, 32:48],
      b_ih[0:16].reshape(1, _D1), b_ih[16:32].reshape(1, _D1),
      b_ih[32:48].reshape(1, _D1),
      b_hh[0:16].reshape(1, _D1), b_hh[16:32].reshape(1, _D1),
      b_hh[32:48].reshape(1, _D1),
  )
  f1 = fc1_w.T
  f1a = f1[0:_D1]
  f1b = f1[_D1:]
  f1bias = fc1_b.reshape(1, _G)
  f2w = fc2_w.T
  f2b = fc2_b.reshape(1, _NC)

  m1 = _matmul_nd(x_p, ggnn_weight[0])
  s1p, degp = _sc_edge_pass(True)(m1, src_p, dst_p, zeros2d, zeros1d, ones128)
  h1, m2, degc = _gru1_call(s1p, degp.reshape(_NCORES, _NPAD, 1), x_p, gru_w,
                            ggnn_weight[1])

  res = _sc_edge_pass(False)(m2, src_p, dst_p, zeros2d, zeros1d, ones128)
  s2p = res[0] if isinstance(res, (list, tuple)) else res
  out = _gru2_call(s2p, degc, h1, batch3, tfidf_vec, f1a, f1b, f1bias, f2w,
                   f2b, gru_w)
  return out


# speedup vs baseline: 42.4133x; 1.3587x over previous
import functools

import jax
import jax.numpy as jnp
from jax import lax
from jax.experimental import pallas as pl
from jax.experimental.pallas import tpu as pltpu
from jax.experimental.pallas import tpu_sc as plsc

_N = 100000
_E = 3200000
_D1 = 16
_G = 64
_NC = 6

_NCORES = 2
_NSUB = 16
_NW = _NCORES * _NSUB
_GB = 128
_GPW = 784
_TOTG = _NW * _GPW
_EPAD = _TOTG * _GB
_KG = 28
_NOUTER = _GPW // _KG

_NPAD = 100096
_TROWS = _NPAD // _NSUB
_M = _NPAD // 8

_GPAD = 72
_NCHUNK = _NPAD // _GB
_BROWS = 800

_RB = 544
_NBLK = _M // _RB


def _sc_edge_pass(with_deg):
  mesh = plsc.VectorSubcoreMesh(core_axis_name="c", subcore_axis_name="s")
  out_type = [jax.ShapeDtypeStruct((_NCORES, _NPAD, _D1), jnp.float32)]
  if with_deg:
    out_type.append(jax.ShapeDtypeStruct((_NCORES * _NPAD,), jnp.float32))
  scratch = [
      pltpu.VMEM((_KG, _GB), jnp.int32),
      pltpu.VMEM((_KG, _GB), jnp.int32),
      pltpu.VMEM((4 * _GB, _D1), jnp.float32),
      pltpu.VMEM((_GB,), jnp.float32),
      pltpu.VMEM((_GB, _D1), jnp.float32),
      pltpu.VMEM((_GB, _D1), jnp.float32),
      pltpu.VMEM((1024,), jnp.float32),
      pltpu.VMEM_SHARED((_NPAD, _D1), jnp.float32),
      pltpu.VMEM_SHARED((_NPAD,), jnp.float32),
      pltpu.SemaphoreType.DMA,
      pltpu.SemaphoreType.DMA,
  ]
  _NCH = 49
  _CTAIL = _TROWS - 48 * _GB

  def body(m_hbm, src_hbm, dst_hbm, z2_hbm, z1_hbm, ones_hbm, *rest):
    if with_deg:
      out_hbm, deg_hbm = rest[0], rest[1]
      scr = rest[2:]
    else:
      out_hbm = rest[0]
      scr = rest[1:]
    (src_v, dst_v, rows, ones_v, stage_a, stage_b, deg_v, acc_sh, deg_sh,
     sem_a, sem_b) = scr
    c = lax.axis_index("c")
    s = lax.axis_index("s")
    wid = c * _NSUB + s

    row0 = s * _TROWS
    pltpu.sync_copy(z2_hbm, stage_a)
    for k in range(48):
      pltpu.sync_copy(stage_a, acc_sh.at[pl.ds(row0 + k * _GB, _GB)])
    pltpu.sync_copy(stage_a.at[pl.ds(0, _CTAIL)],
                    acc_sh.at[pl.ds(row0 + 48 * _GB, _CTAIL)])
    if with_deg:
      pltpu.sync_copy(z1_hbm, deg_v)
      for k in range(6):
        pltpu.sync_copy(deg_v, deg_sh.at[pl.ds(row0 + k * 1024, 1024)])
      pltpu.sync_copy(deg_v.at[pl.ds(0, _CTAIL)],
                      deg_sh.at[pl.ds(row0 + 6144, _CTAIL)])
    pltpu.sync_copy(ones_hbm, ones_v)
    plsc.subcore_barrier()

    sems = (sem_a, sem_b)
    g_base = wid * _GPW

    def fire_pair(first_g, half, sem):
      for b in range(2):
        pltpu.async_copy(
            m_hbm.at[src_v.at[first_g + b]],
            rows.at[pl.ds((half * 2 + b) * _GB, _GB)],
            sem,
        )

    def outer(blk, carry):
      g0 = g_base + blk * _KG
      pltpu.sync_copy(src_hbm.at[pl.ds(g0, _KG)], src_v)
      pltpu.sync_copy(dst_hbm.at[pl.ds(g0, _KG)], dst_v)
      fire_pair(0, 0, sems[0])

      def inner(ci, carry2):
        for half in range(2):
          qbase = ci * 4 + half * 2
          nbase = qbase + 2

          @pl.when(nbase < _KG)
          def _():
            fire_pair(nbase, 1 - half, sems[1 - half])

          pltpu.make_async_copy(
              m_hbm.at[pl.ds(0, 2 * _GB)],
              rows.at[pl.ds(half * 2 * _GB, 2 * _GB)],
              sems[half],
          ).wait()
          for b in range(2):
            g = qbase + b
            rsl = rows.at[pl.ds((half * 2 + b) * _GB, _GB)]
            pltpu.sync_copy(rsl, acc_sh.at[dst_v.at[g]], add=True)
            if with_deg:
              pltpu.sync_copy(ones_v, deg_sh.at[dst_v.at[g]], add=True)
        return carry2

      lax.fori_loop(0, _KG // 4, inner, 0)
      return carry

    lax.fori_loop(0, _NOUTER, outer, 0)
    plsc.subcore_barrier()

    for k in range(_NCH):
      nrows = _GB if k < _NCH - 1 else _CTAIL
      buf = stage_a if k % 2 == 0 else stage_b
      sem = sems[k % 2]
      off = row0 + k * _GB
      if k >= 2:
        pltpu.make_async_copy(
            buf, out_hbm.at[c, pl.ds(row0, _GB)], sem).wait()
      pltpu.sync_copy(acc_sh.at[pl.ds(off, nrows)], buf.at[pl.ds(0, nrows)])
      pltpu.async_copy(buf.at[pl.ds(0, nrows)],
                       out_hbm.at[c, pl.ds(off, nrows)], sem)
    pltpu.make_async_copy(
        stage_b, out_hbm.at[c, pl.ds(row0, _GB)], sems[1]).wait()
    pltpu.make_async_copy(
        stage_a.at[pl.ds(0, _CTAIL)], out_hbm.at[c, pl.ds(row0, _CTAIL)],
        sems[0]).wait()
    if with_deg:
      for k in range(6):
        pltpu.sync_copy(deg_sh.at[pl.ds(row0 + k * 1024, 1024)], deg_v)
        pltpu.sync_copy(
            deg_v, deg_hbm.at[pl.ds(c * _NPAD + row0 + k * 1024, 1024)])
      pltpu.sync_copy(deg_sh.at[pl.ds(row0 + 6144, _CTAIL)],
                      deg_v.at[pl.ds(0, _CTAIL)])
      pltpu.sync_copy(deg_v.at[pl.ds(0, _CTAIL)],
                      deg_hbm.at[pl.ds(c * _NPAD + row0 + 6144, _CTAIL)])

  return pl.kernel(
      body, mesh=mesh, out_type=out_type, scratch_types=scratch,
      compiler_params=pltpu.CompilerParams(use_tc_tiling_on_sc=False))


def _sc_pool():
  mesh = plsc.VectorSubcoreMesh(core_axis_name="c", subcore_axis_name="s")
  out_type = [
      jax.ShapeDtypeStruct((_NCORES, _GPAD, _D1), jnp.float32),
      jax.ShapeDtypeStruct((_NCORES, _GPAD, _D1), jnp.float32),
  ]
  scratch = [
      pltpu.VMEM((25, _GB), jnp.int32),
      pltpu.VMEM((_GB, _D1), jnp.float32),
      pltpu.VMEM((_GB, _D1), jnp.float32),
      pltpu.VMEM((_GPAD, _D1), jnp.float32),
      pltpu.VMEM_SHARED((_GPAD, _D1), jnp.float32),
      pltpu.VMEM_SHARED((_GPAD, _D1), jnp.float32),
  ]

  def body(h_hbm, bat_hbm, z2_hbm, ones_hbm, pool_hbm, cnt_hbm,
           bat_v, val_v, ones_v, stage_v, pool_sh, cnt_sh):
    c = lax.axis_index("c")
    s = lax.axis_index("s")
    wid = c * _NSUB + s
    c0 = 24 * wid + jnp.minimum(wid, 14)
    nc = 24 + jnp.where(wid < 14, 1, 0)

    @pl.when(s == 0)
    def _():
      pltpu.sync_copy(z2_hbm.at[pl.ds(0, _GPAD)], stage_v)
      pltpu.sync_copy(stage_v, pool_sh)
      pltpu.sync_copy(stage_v, cnt_sh)

    pltpu.sync_copy(ones_hbm, ones_v)
    pltpu.sync_copy(bat_hbm.at[pl.ds(c0, 25)], bat_v)
    plsc.subcore_barrier()

    def chunk(k, carry):
      n0 = (c0 + k) * _GB
      pltpu.sync_copy(h_hbm.at[pl.ds(n0, _GB)], val_v)
      pltpu.sync_copy(val_v, pool_sh.at[bat_v.at[k]], add=True)
      pltpu.sync_copy(ones_v, cnt_sh.at[bat_v.at[k]], add=True)
      return carry

    lax.fori_loop(0, nc, chunk, 0)
    plsc.subcore_barrier()

    @pl.when(s == 0)
    def _():
      pltpu.sync_copy(pool_sh, stage_v)
      pltpu.sync_copy(stage_v, pool_hbm.at[c])
      pltpu.sync_copy(cnt_sh, stage_v)
      pltpu.sync_copy(stage_v, cnt_hbm.at[c])

  return pl.kernel(
      body, mesh=mesh, out_type=out_type, scratch_types=scratch,
      compiler_params=pltpu.CompilerParams(use_tc_tiling_on_sc=False))


def _mm_kernel(x_ref, w_ref, o_ref):
  o_ref[...] = jnp.dot(x_ref[...], w_ref[...],
                       preferred_element_type=jnp.float32)


def _matmul_packed(x, w):
  return pl.pallas_call(
      _mm_kernel,
      grid=(_NBLK,),
      in_specs=[
          pl.BlockSpec((_RB, 128), lambda i: (i, 0)),
          pl.BlockSpec((128, 128), lambda i: (0, 0)),
      ],
      out_specs=pl.BlockSpec((_RB, 128), lambda i: (i, 0)),
      out_shape=jax.ShapeDtypeStruct((_M, 128), jnp.float32),
  )(x, w)


def _gru_block(agg, h, w):
  (wir, wiz, win, whr, whz, whn, bir, biz, bin_, bhr, bhz, bhn) = w
  dot = functools.partial(jnp.dot, preferred_element_type=jnp.float32)
  r = jax.nn.sigmoid(dot(agg, wir) + bir + dot(h, whr) + bhr)
  z = jax.nn.sigmoid(dot(agg, wiz) + biz + dot(h, whz) + bhz)
  n = jnp.tanh(dot(agg, win) + bin_ + r * (dot(h, whn) + bhn))
  return (1.0 - z) * n + z * h


_GRU_W_SPECS = (
    [pl.BlockSpec((128, 128), lambda i: (0, 0))] * 6
    + [pl.BlockSpec((1, 128), lambda i: (0, 0))] * 6
)


def _gru1_kernel(s_ref, rd_ref, x_ref, *rest):
  w = [r[...] for r in rest[:12]]
  w1 = rest[12]
  h1_ref, m2_ref = rest[13], rest[14]
  agg = (s_ref[0] + s_ref[1]) * rd_ref[...]
  h1 = _gru_block(agg, x_ref[...], w)
  h1_ref[...] = h1
  m2_ref[...] = jnp.dot(h1, w1[...], preferred_element_type=jnp.float32)


def _gru1_call(s1p, rdeg, x, gru_w, w1k):
  return pl.pallas_call(
      _gru1_kernel,
      grid=(_NBLK,),
      in_specs=[
          pl.BlockSpec((_NCORES, _RB, 128), lambda i: (0, i, 0)),
          pl.BlockSpec((_RB, 128), lambda i: (i, 0)),
          pl.BlockSpec((_RB, 128), lambda i: (i, 0)),
      ] + _GRU_W_SPECS + [pl.BlockSpec((128, 128), lambda i: (0, 0))],
      out_specs=[
          pl.BlockSpec((_RB, 128), lambda i: (i, 0)),
          pl.BlockSpec((_RB, 128), lambda i: (i, 0)),
      ],
      out_shape=[
          jax.ShapeDtypeStruct((_M, 128), jnp.float32),
          jax.ShapeDtypeStruct((_M, 128), jnp.float32),
      ],
  )(s1p, rdeg, x, *gru_w, w1k)


def _gru2_kernel(s_ref, rd_ref, h_ref, *rest):
  w = [r[...] for r in rest[:12]]
  hr_ref = rest[12]
  agg = (s_ref[0] + s_ref[1]) * rd_ref[...]
  h2 = _gru_block(agg, h_ref[...], w)
  hr_ref[...] = jnp.maximum(h2, 0.0)


def _gru2_call(s2p, rdeg, h1, gru_w):
  return pl.pallas_call(
      _gru2_kernel,
      grid=(_NBLK,),
      in_specs=[
          pl.BlockSpec((_NCORES, _RB, 128), lambda i: (0, i, 0)),
          pl.BlockSpec((_RB, 128), lambda i: (i, 0)),
          pl.BlockSpec((_RB, 128), lambda i: (i, 0)),
      ] + _GRU_W_SPECS,
      out_specs=pl.BlockSpec((_RB, 128), lambda i: (i, 0)),
      out_shape=jax.ShapeDtypeStruct((_M, 128), jnp.float32),
  )(s2p, rdeg, h1, *gru_w)


def _head_kernel(p_ref, c_ref, tfidf_ref, f1a_ref, f1b_ref, f1bias_ref,
                 f2w_ref, f2b_ref, out_ref):
  dot = functools.partial(jnp.dot, preferred_element_type=jnp.float32)
  pooled = (p_ref[0] + p_ref[1])[0:_G, :]
  cnt = (c_ref[0] + c_ref[1])[0:_G, :]
  mean = pooled / jnp.maximum(cnt, 1.0)
  a = jnp.maximum(
      dot(mean, f1a_ref[...]) + dot(tfidf_ref[...], f1b_ref[...])
      + f1bias_ref[...], 0.0)
  logits = dot(a, f2w_ref[...]) + f2b_ref[...]
  mx = jnp.max(logits, axis=1, keepdims=True)
  lse = jnp.log(jnp.sum(jnp.exp(logits - mx), axis=1, keepdims=True)) + mx
  out_ref[...] = logits - lse


def _head_call(pool2, cnt2, tfidf, f1a, f1b, f1bias, f2w, f2b):
  return pl.pallas_call(
      _head_kernel,
      out_shape=jax.ShapeDtypeStruct((_G, _NC), jnp.float32),
  )(pool2, cnt2, tfidf, f1a, f1b, f1bias, f2w, f2b)


def _kron8(w):
  return jnp.kron(jnp.eye(8, dtype=w.dtype), w)


def _tile8(b):
  return jnp.tile(b, 8).reshape(1, 128)


def kernel(x, edge_index, batch, tfidf_vec, ggnn_weight, w_ih, w_hh, b_ih,
           b_hh, fc1_w, fc1_b, fc2_w, fc2_b):
  f32 = jnp.float32
  src = edge_index[0]
  dst = edge_index[1]
  pad = _EPAD - _E
  src_p = jnp.concatenate([src, jnp.zeros((pad,), jnp.int32)]).reshape(
      _TOTG, _GB)
  dst_p = jnp.concatenate(
      [dst, _N + (jnp.arange(pad, dtype=jnp.int32) % (_NPAD - _N))]).reshape(
          _TOTG, _GB)
  zeros2d = jnp.zeros((_GB, _D1), f32)
  zeros1d = jnp.zeros((1024,), f32)
  ones128 = jnp.ones((_GB,), f32)
  ones2d = jnp.ones((_GB, _D1), f32)
  xp = jnp.pad(x, ((0, _NPAD - _N), (0, 0))).reshape(_M, 128)
  npb = _BROWS * _GB - _N
  bat_p = jnp.concatenate(
      [batch, _G + (jnp.arange(npb, dtype=jnp.int32) % 8)]).reshape(
          _BROWS, _GB)

  wi = w_ih.T
  wh = w_hh.T
  gru_w = (
      _kron8(wi[:, 0:16]), _kron8(wi[:, 16:32]), _kron8(wi[:, 32:48]),
      _kron8(wh[:, 0:16]), _kron8(wh[:, 16:32]), _kron8(wh[:, 32:48]),
      _tile8(b_ih[0:16]), _tile8(b_ih[16:32]), _tile8(b_ih[32:48]),
      _tile8(b_hh[0:16]), _tile8(b_hh[16:32]), _tile8(b_hh[32:48]),
  )
  w0k = _kron8(ggnn_weight[0])
  w1k = _kron8(ggnn_weight[1])
  f1 = fc1_w.T
  f1a = f1[0:_D1]
  f1b = f1[_D1:]
  f1bias = fc1_b.reshape(1, _G)
  f2w = fc2_w.T
  f2b = fc2_b.reshape(1, _NC)

  m1 = _matmul_packed(xp, w0k)
  s1p, degp = _sc_edge_pass(True)(
      m1.reshape(_NPAD, _D1), src_p, dst_p, zeros2d, zeros1d, ones128)
  rdeg = jnp.repeat(
      1.0 / jnp.maximum(degp[:_NPAD] + degp[_NPAD:], 1.0), _D1).reshape(
          _M, 128)
  h1, m2 = _gru1_call(s1p.reshape(_NCORES, _M, 128), rdeg, xp, gru_w, w1k)

  res = _sc_edge_pass(False)(
      m2.reshape(_NPAD, _D1), src_p, dst_p, zeros2d, zeros1d, ones128)
  s2p = res[0] if isinstance(res, (list, tuple)) else res
  h2r = _gru2_call(s2p.reshape(_NCORES, _M, 128), rdeg, h1, gru_w)
  pool2, cnt2 = _sc_pool()(h2r.reshape(_NPAD, _D1), bat_p, zeros2d, ones2d)
  return _head_call(pool2, cnt2, tfidf_vec, f1a, f1b, f1bias, f2w, f2b)
